# baseline (speedup 1.0000x reference)
import functools

import jax
import jax.numpy as jnp
from jax import lax
from jax.experimental import pallas as pl
from jax.experimental.pallas import tpu as pltpu
from jax.experimental.pallas import tpu_sc as plsc

B, C, H, W = 8, 384, 224, 224
HID, E, K = 96, 64, 2
S = H * W
HB = 28

WT, CT, SB, LN = 28, 3, 8, 128
HWT = 14
NACC = CT * (LN // 16)

H_SC = 224
H_TC = H - H_SC
WPB = 4
HPW = H_SC // WPB


def _sc_body(x_ref, out_ref, buf, accv, sem0, sem1):
    cid = lax.axis_index("c")
    sid = lax.axis_index("s")
    wid = sid * 2 + cid
    b = wid // WPB
    q = lax.rem(wid, WPB)
    h0 = H_TC + q * HPW
    sems = (sem0, sem1)

    def src(h, p):
        return x_ref.at[b, h, pl.ds(p * HWT, HWT)]

    pltpu.async_copy(src(h0, 0), buf.at[0], sem0)
    pltpu.async_copy(src(h0, 1), buf.at[1], sem1)

    def consume(p, accs):
        def inner(wi, accs):
            accs = list(accs)
            for ct in range(CT):
                for sb in range(SB):
                    for lg in range(LN // 16):
                        j = ct * (LN // 16) + lg
                        v = buf[p, wi, ct, sb, pl.ds(lg * 16, 16)]
                        accs[j] = accs[j] + v
            return tuple(accs)
        return lax.fori_loop(0, HWT, inner, accs)

    def row(t, accs):
        h = h0 + t
        for p in range(2):
            pltpu.make_async_copy(src(h0, p), buf.at[p], sems[p]).wait()
            accs = consume(p, accs)

            @pl.when(t + 1 < HPW)
            def _next(p=p, h=h):
                pltpu.async_copy(src(h + 1, p), buf.at[p], sems[p])
        return accs

    accs = tuple(jnp.zeros((16,), jnp.float32) for _ in range(NACC))
    accs = lax.fori_loop(0, HPW, row, accs)
    for j in range(NACC):
        accv[j, :] = accs[j]
    pltpu.sync_copy(accv, out_ref.at[b, q])


def _sc_reduce(xsc):
    mesh = plsc.VectorSubcoreMesh(core_axis_name="c", subcore_axis_name="s")
    return pl.kernel(
        _sc_body,
        out_type=jax.ShapeDtypeStruct((B, WPB, NACC, 16), jnp.float32),
        mesh=mesh,
        scratch_types=[
            pltpu.VMEM((2, HWT, CT, SB, LN), jnp.float32),
            pltpu.VMEM((NACC, 16), jnp.float32),
            pltpu.SemaphoreType.DMA,
            pltpu.SemaphoreType.DMA,
        ],
    )(xsc)


def _tc_body(x_ref, part_ref):
    hb = pl.program_id(1)
    psum = jnp.sum(x_ref[...], axis=(1, 2))
    b = pl.program_id(0)

    @pl.when(hb == 0)
    def _init():
        part_ref[pl.ds(b, 1), :] = psum

    @pl.when(hb != 0)
    def _acc():
        part_ref[pl.ds(b, 1), :] += psum


def _tc_reduce(xt):
    return pl.pallas_call(
        _tc_body,
        grid=(B, H_TC // HB),
        in_specs=[pl.BlockSpec((1, HB, W, C), lambda b, hb: (b, hb, 0, 0))],
        out_specs=pl.BlockSpec((B, C), lambda b, hb: (0, 0)),
        out_shape=jax.ShapeDtypeStruct((B, C), jnp.float32),
        compiler_params=pltpu.CompilerParams(
            dimension_semantics=("arbitrary", "arbitrary"),
        ),
    )(xt)


def _router_body(tc_ref, sc_ref, w1_ref, b1_ref, w2_ref, b2_ref,
                 idx_ref, val_ref, probs_ref):
    h = (tc_ref[...] + jnp.sum(sc_ref[...], axis=1)) * (1.0 / S)
    hid = lax.dot_general(h, w1_ref[...], (((1,), (1,)), ((), ())),
                          preferred_element_type=jnp.float32)
    hid = jnp.maximum(hid + b1_ref[...], 0.0)
    logits = lax.dot_general(hid, w2_ref[...], (((1,), (1,)), ((), ())),
                             preferred_element_type=jnp.float32)
    logits = logits + b2_ref[...]
    m = jnp.max(logits, axis=1, keepdims=True)
    e = jnp.exp(logits - m)
    p = e / jnp.sum(e, axis=1, keepdims=True)
    probs_ref[...] = p
    iota = lax.broadcasted_iota(jnp.int32, p.shape, 1)
    m1 = jnp.max(p, axis=1, keepdims=True)
    i1 = jnp.min(jnp.where(p == m1, iota, E), axis=1, keepdims=True)
    p2 = jnp.where(iota == i1, -jnp.inf, p)
    m2 = jnp.max(p2, axis=1, keepdims=True)
    i2 = jnp.min(jnp.where(p2 == m2, iota, E), axis=1, keepdims=True)
    val_ref[...] = jnp.concatenate([m1, m2], axis=1)
    idx_ref[...] = jnp.concatenate([i1, i2], axis=1)


def _router(tc_part, sc_part, W1, b1r, W2, b2r):
    return pl.pallas_call(
        _router_body,
        out_shape=[
            jax.ShapeDtypeStruct((B, K), jnp.int32),
            jax.ShapeDtypeStruct((B, K), jnp.float32),
            jax.ShapeDtypeStruct((B, E), jnp.float32),
        ],
    )(tc_part, sc_part, W1, b1r, W2, b2r)


@jax.jit
def kernel(x, W1, b1, W2, b2):
    xt = jnp.transpose(x, (0, 2, 3, 1))
    xsc = xt.reshape(B, H, WT, SB, CT, LN).transpose(0, 1, 2, 4, 3, 5)
    b1r = b1.reshape(1, HID)
    b2r = b2.reshape(1, E)

    if H_TC > 0:
        tc_part = _tc_reduce(xt)
    else:
        tc_part = jnp.zeros((B, C), jnp.float32)
    if H_SC > 0:
        sc_part = _sc_reduce(xsc).reshape(B, WPB, C)
    else:
        sc_part = jnp.zeros((B, WPB, C), jnp.float32)

    topk_idx, topk_val, probs = _router(tc_part, sc_part, W1, b1r, W2, b2r)
    return (topk_idx, topk_val, probs)


# speedup vs baseline: 2.1303x; 2.1303x over previous
import functools

import jax
import jax.numpy as jnp
from jax import lax
from jax.experimental import pallas as pl
from jax.experimental.pallas import tpu as pltpu
from jax.experimental.pallas import tpu_sc as plsc

B, C, H, W = 8, 384, 224, 224
HID, E, K = 96, 64, 2
S = H * W
HB = 28

WT, CT, SB, LN = 28, 3, 8, 128
HWT = 14
NACC = CT * (LN // 16)

H_SC = 56
H_TC = H - H_SC
WPB = 4
HPW = H_SC // WPB


def _sc_body(x_ref, out_ref, buf, accv, sem0, sem1):
    cid = lax.axis_index("c")
    sid = lax.axis_index("s")
    wid = sid * 2 + cid
    b = wid // WPB
    q = lax.rem(wid, WPB)
    h0 = H_TC + q * HPW
    sems = (sem0, sem1)

    def src(h, p):
        return x_ref.at[b, h, pl.ds(p * HWT, HWT)]

    pltpu.async_copy(src(h0, 0), buf.at[0], sem0)
    pltpu.async_copy(src(h0, 1), buf.at[1], sem1)

    def consume(p, accs):
        def inner(wi, accs):
            accs = list(accs)
            for ct in range(CT):
                for sb in range(SB):
                    for lg in range(LN // 16):
                        j = ct * (LN // 16) + lg
                        v = buf[p, wi, ct, sb, pl.ds(lg * 16, 16)]
                        accs[j] = accs[j] + v
            return tuple(accs)
        return lax.fori_loop(0, HWT, inner, accs)

    def row(t, accs):
        h = h0 + t
        for p in range(2):
            pltpu.make_async_copy(src(h0, p), buf.at[p], sems[p]).wait()
            accs = consume(p, accs)

            @pl.when(t + 1 < HPW)
            def _next(p=p, h=h):
                pltpu.async_copy(src(h + 1, p), buf.at[p], sems[p])
        return accs

    accs = tuple(jnp.zeros((16,), jnp.float32) for _ in range(NACC))
    accs = lax.fori_loop(0, HPW, row, accs)
    for j in range(NACC):
        accv[j, :] = accs[j]
    pltpu.sync_copy(accv, out_ref.at[b, q])


def _sc_reduce(xsc):
    mesh = plsc.VectorSubcoreMesh(core_axis_name="c", subcore_axis_name="s")
    return pl.kernel(
        _sc_body,
        out_type=jax.ShapeDtypeStruct((B, WPB, NACC, 16), jnp.float32),
        mesh=mesh,
        scratch_types=[
            pltpu.VMEM((2, HWT, CT, SB, LN), jnp.float32),
            pltpu.VMEM((NACC, 16), jnp.float32),
            pltpu.SemaphoreType.DMA,
            pltpu.SemaphoreType.DMA,
        ],
    )(xsc)


def _tc_body(x_ref, part_ref):
    hb = pl.program_id(1)
    psum = jnp.sum(x_ref[...], axis=(1, 2))
    b = pl.program_id(0)

    @pl.when(hb == 0)
    def _init():
        part_ref[pl.ds(b, 1), :] = psum

    @pl.when(hb != 0)
    def _acc():
        part_ref[pl.ds(b, 1), :] += psum


def _tc_reduce(xt):
    return pl.pallas_call(
        _tc_body,
        grid=(B, H_TC // HB),
        in_specs=[pl.BlockSpec((1, HB, W, C), lambda b, hb: (b, hb, 0, 0))],
        out_specs=pl.BlockSpec((B, C), lambda b, hb: (0, 0)),
        out_shape=jax.ShapeDtypeStruct((B, C), jnp.float32),
        compiler_params=pltpu.CompilerParams(
            dimension_semantics=("arbitrary", "arbitrary"),
        ),
    )(xt)


def _router_body(tc_ref, sc_ref, w1_ref, b1_ref, w2_ref, b2_ref,
                 idx_ref, val_ref, probs_ref):
    h = (tc_ref[...] + jnp.sum(sc_ref[...], axis=1)) * (1.0 / S)
    hid = lax.dot_general(h, w1_ref[...], (((1,), (1,)), ((), ())),
                          preferred_element_type=jnp.float32)
    hid = jnp.maximum(hid + b1_ref[...], 0.0)
    logits = lax.dot_general(hid, w2_ref[...], (((1,), (1,)), ((), ())),
                             preferred_element_type=jnp.float32)
    logits = logits + b2_ref[...]
    m = jnp.max(logits, axis=1, keepdims=True)
    e = jnp.exp(logits - m)
    p = e / jnp.sum(e, axis=1, keepdims=True)
    probs_ref[...] = p
    iota = lax.broadcasted_iota(jnp.int32, p.shape, 1)
    m1 = jnp.max(p, axis=1, keepdims=True)
    i1 = jnp.min(jnp.where(p == m1, iota, E), axis=1, keepdims=True)
    p2 = jnp.where(iota == i1, -jnp.inf, p)
    m2 = jnp.max(p2, axis=1, keepdims=True)
    i2 = jnp.min(jnp.where(p2 == m2, iota, E), axis=1, keepdims=True)
    val_ref[...] = jnp.concatenate([m1, m2], axis=1)
    idx_ref[...] = jnp.concatenate([i1, i2], axis=1)


def _router(tc_part, sc_part, W1, b1r, W2, b2r):
    return pl.pallas_call(
        _router_body,
        out_shape=[
            jax.ShapeDtypeStruct((B, K), jnp.int32),
            jax.ShapeDtypeStruct((B, K), jnp.float32),
            jax.ShapeDtypeStruct((B, E), jnp.float32),
        ],
    )(tc_part, sc_part, W1, b1r, W2, b2r)


@jax.jit
def kernel(x, W1, b1, W2, b2):
    xt = jnp.transpose(x, (0, 2, 3, 1))
    xsc = xt.reshape(B, H, WT, SB, CT, LN).transpose(0, 1, 2, 4, 3, 5)
    b1r = b1.reshape(1, HID)
    b2r = b2.reshape(1, E)

    if H_TC > 0:
        tc_part = _tc_reduce(xt)
    else:
        tc_part = jnp.zeros((B, C), jnp.float32)
    if H_SC > 0:
        sc_part = _sc_reduce(xsc).reshape(B, WPB, C)
    else:
        sc_part = jnp.zeros((B, WPB, C), jnp.float32)

    topk_idx, topk_val, probs = _router(tc_part, sc_part, W1, b1r, W2, b2r)
    return (topk_idx, topk_val, probs)


# speedup vs baseline: 2.3876x; 1.1208x over previous
import functools

import jax
import jax.numpy as jnp
from jax import lax
from jax.experimental import pallas as pl
from jax.experimental.pallas import tpu as pltpu

B, C, H, W = 8, 384, 224, 224
HID, E, K = 96, 64, 2
S = H * W
HB = 56
NH = H // HB


def _body(x_ref, w1_ref, b1_ref, w2_ref, b2_ref,
          idx_ref, val_ref, probs_ref, part_ref):
    b = pl.program_id(0)
    hb = pl.program_id(1)
    psum = jnp.sum(x_ref[...], axis=(1, 2))

    @pl.when(hb == 0)
    def _init():
        part_ref[pl.ds(b, 1), :] = psum

    @pl.when(hb != 0)
    def _acc():
        part_ref[pl.ds(b, 1), :] += psum

    @pl.when((b == B - 1) & (hb == NH - 1))
    def _router():
        h = part_ref[...] * (1.0 / S)
        hid = lax.dot_general(h, w1_ref[...], (((1,), (1,)), ((), ())),
                              preferred_element_type=jnp.float32)
        hid = jnp.maximum(hid + b1_ref[...], 0.0)
        logits = lax.dot_general(hid, w2_ref[...], (((1,), (1,)), ((), ())),
                                 preferred_element_type=jnp.float32)
        logits = logits + b2_ref[...]
        m = jnp.max(logits, axis=1, keepdims=True)
        e = jnp.exp(logits - m)
        p = e / jnp.sum(e, axis=1, keepdims=True)
        probs_ref[...] = p
        iota = lax.broadcasted_iota(jnp.int32, p.shape, 1)
        m1 = jnp.max(p, axis=1, keepdims=True)
        i1 = jnp.min(jnp.where(p == m1, iota, E), axis=1, keepdims=True)
        p2 = jnp.where(iota == i1, -jnp.inf, p)
        m2 = jnp.max(p2, axis=1, keepdims=True)
        i2 = jnp.min(jnp.where(p2 == m2, iota, E), axis=1, keepdims=True)
        val_ref[...] = jnp.concatenate([m1, m2], axis=1)
        idx_ref[...] = jnp.concatenate([i1, i2], axis=1)


@jax.jit
def kernel(x, W1, b1, W2, b2):
    xt = jnp.transpose(x, (0, 2, 3, 1))
    b1r = b1.reshape(1, HID)
    b2r = b2.reshape(1, E)

    out = pl.pallas_call(
        _body,
        grid=(B, NH),
        in_specs=[pl.BlockSpec((1, HB, W, C), lambda b, hb: (b, hb, 0, 0)),
                  pl.BlockSpec((HID, C), lambda b, hb: (0, 0)),
                  pl.BlockSpec((1, HID), lambda b, hb: (0, 0)),
                  pl.BlockSpec((E, HID), lambda b, hb: (0, 0)),
                  pl.BlockSpec((1, E), lambda b, hb: (0, 0))],
        out_specs=[
            pl.BlockSpec((B, K), lambda b, hb: (0, 0)),
            pl.BlockSpec((B, K), lambda b, hb: (0, 0)),
            pl.BlockSpec((B, E), lambda b, hb: (0, 0)),
        ],
        out_shape=[
            jax.ShapeDtypeStruct((B, K), jnp.int32),
            jax.ShapeDtypeStruct((B, K), jnp.float32),
            jax.ShapeDtypeStruct((B, E), jnp.float32),
        ],
        scratch_shapes=[pltpu.VMEM((B, C), jnp.float32)],
        compiler_params=pltpu.CompilerParams(
            dimension_semantics=("arbitrary", "arbitrary"),
        ),
    )(xt, W1, b1r, W2, b2r)
    topk_idx, topk_val, probs = out
    return (topk_idx, topk_val, probs)
